# baseline (speedup 1.0000x reference)
import functools

import jax
import jax.numpy as jnp
from jax import lax
from jax.experimental import pallas as pl
from jax.experimental.pallas import tpu as pltpu
from jax.experimental.pallas import tpu_sc as plsc

MAX_REL = 128
NUM_HEADS = 8
D_K = 64
L_STATIC = 512
D = NUM_HEADS * D_K
S_ROWS = 1024
NS = 16
HDB = D // NS
OUT_ROW = L_STATIC * D
SLAB = HDB * L_STATIC


def _sc_body(tk_hbm, tv_hbm, outk_hbm, outv_hbm,
             idx2d, gown, clamp_lo, clamp_hi, bufa, bufb, bufc, bufd,
             sems, sema, semb, semc, semd):
    cid = lax.axis_index("c")
    sid = lax.axis_index("s")
    hd0 = sid * HDB

    def run(tbl, out):
        pltpu.async_copy(tbl.at[pl.ds(256 * D, D)], clamp_hi, sems).wait()
        pltpu.async_copy(tbl.at[pl.ds(0, D)], clamp_lo, sems).wait()

        def mid_group(g, carry):
            hl = g // 2
            p = g % 2
            for m in range(8):
                vec = ((255 - 128 * p - 16 * m) * D + hd0 + hl
                       - lax.iota(jnp.int32, 16) * D)
                idx2d[g, pl.ds(16 * m, 16)] = vec
            pltpu.async_copy(
                tbl.at[idx2d.at[g]],
                gown.at[pl.ds(hl * S_ROWS + 384 + 128 * p, 128)], sems)
            return carry

        lax.fori_loop(0, HDB * 2, mid_group, 0)
        for _ in range(HDB * 2):
            pltpu.make_async_copy(
                tbl.at[pl.ds(0, 128)], gown.at[pl.ds(0, 128)], sems).wait()

        for hl in range(HDB):
            chunk = pl.ds(hd0 + (hl // 16) * 16, 16)
            vhi = jnp.full((16,), clamp_hi[chunk][hl % 16], jnp.float32)
            vlo = jnp.full((16,), clamp_lo[chunk][hl % 16], jnp.float32)

            def clamp_fill(m, carry, hl=hl, vhi=vhi, vlo=vlo):
                gown[pl.ds(hl * S_ROWS + 16 * m, 16)] = vhi
                gown[pl.ds(hl * S_ROWS + 640 + 16 * m, 16)] = vlo
                return carry

            lax.fori_loop(0, 24, clamp_fill, 0)

        ring = ((bufa, sema), (bufb, semb), (bufc, semc), (bufd, semd))

        def emit(t, carry):
            for par, (buf, sem) in enumerate(ring):
                i = 4 * t + par
                o = 511 - i

                @pl.when(t > 0)
                def _(buf=buf, sem=sem):
                    pltpu.make_async_copy(
                        buf, out.at[pl.ds(hd0 * L_STATIC, SLAB)], sem).wait()

                @plsc.parallel_loop(0, HDB * 32, unroll=16)
                def fill_row(q, buf=buf):
                    buf[pl.ds(16 * q, 16)] = (
                        gown[pl.ds(16 * q + 512 * (q // 32) + o, 16)])
                dst = pl.multiple_of(i * OUT_ROW + hd0 * L_STATIC, D)
                pltpu.async_copy(buf, out.at[pl.ds(dst, SLAB)], sem)
            return carry

        lax.fori_loop(0, L_STATIC // 4, emit, 0)
        for buf, sem in ring:
            pltpu.make_async_copy(
                buf, out.at[pl.ds(hd0 * L_STATIC, SLAB)], sem).wait()

    @pl.when(cid == 0)
    def _():
        run(tk_hbm, outk_hbm)

    @pl.when(cid != 0)
    def _():
        run(tv_hbm, outv_hbm)


@jax.jit
def _rel_emb(tk, tv):
    mesh = plsc.VectorSubcoreMesh(core_axis_name="c", subcore_axis_name="s")
    f = functools.partial(
        pl.kernel,
        mesh=mesh,
        out_type=(
            jax.ShapeDtypeStruct((L_STATIC * OUT_ROW,), jnp.float32),
            jax.ShapeDtypeStruct((L_STATIC * OUT_ROW,), jnp.float32),
        ),
        scratch_types=[
            pltpu.VMEM((HDB * 2, 128), jnp.int32),
            pltpu.VMEM((HDB * S_ROWS,), jnp.float32),
            pltpu.VMEM((D,), jnp.float32),
            pltpu.VMEM((D,), jnp.float32),
            pltpu.VMEM((SLAB,), jnp.float32),
            pltpu.VMEM((SLAB,), jnp.float32),
            pltpu.VMEM((SLAB,), jnp.float32),
            pltpu.VMEM((SLAB,), jnp.float32),
            pltpu.SemaphoreType.DMA,
            pltpu.SemaphoreType.DMA,
            pltpu.SemaphoreType.DMA,
            pltpu.SemaphoreType.DMA,
            pltpu.SemaphoreType.DMA,
        ],
    )(_sc_body)
    return f(tk, tv)


def kernel(rel_pos_emb_k, rel_pos_emb_v, L):
    tk = rel_pos_emb_k.reshape(-1)
    tv = rel_pos_emb_v.reshape(-1)
    outk, outv = _rel_emb(tk, tv)
    outk = outk.reshape(L_STATIC, NUM_HEADS, D_K, L_STATIC).transpose(0, 3, 1, 2)
    outv = outv.reshape(L_STATIC, NUM_HEADS, D_K, L_STATIC).transpose(0, 3, 1, 2)
    return (outk, outv)


# speedup vs baseline: 1.3253x; 1.3253x over previous
import functools

import jax
import jax.numpy as jnp
from jax import lax
from jax.experimental import pallas as pl
from jax.experimental.pallas import tpu as pltpu
from jax.experimental.pallas import tpu_sc as plsc

MAX_REL = 128
NUM_HEADS = 8
D_K = 64
L_STATIC = 512
D = NUM_HEADS * D_K
S_ROWS = 1024
NS = 16
HDB = D // NS
OUT_ROW = L_STATIC * D
SLAB = HDB * L_STATIC


def _sc_body(tk_hbm, tv_hbm, outk_hbm, outv_hbm,
             idx2d, gown, clamp_lo, clamp_hi, bufa, bufb, bufc, bufd,
             cl_sh, cr_sh, sems, sema, semb, semc, semd, semc2):
    cid = lax.axis_index("c")
    sid = lax.axis_index("s")
    hd0 = sid * HDB

    def run(tbl, out):
        pltpu.async_copy(tbl.at[pl.ds(256 * D, D)], clamp_hi, sems).wait()
        pltpu.async_copy(tbl.at[pl.ds(0, D)], clamp_lo, sems).wait()

        def mid_group(g, carry):
            hl = g // 2
            p = g % 2
            for m in range(8):
                vec = ((255 - 128 * p - 16 * m) * D + hd0 + hl
                       - lax.iota(jnp.int32, 16) * D)
                idx2d[g, pl.ds(16 * m, 16)] = vec
            pltpu.async_copy(
                tbl.at[idx2d.at[g]],
                gown.at[pl.ds(hl * S_ROWS + 384 + 128 * p, 128)], sems)
            return carry

        lax.fori_loop(0, HDB * 2, mid_group, 0)
        for _ in range(HDB * 2):
            pltpu.make_async_copy(
                tbl.at[pl.ds(0, 128)], gown.at[pl.ds(0, 128)], sems).wait()

        for hl in range(HDB):
            chunk = pl.ds(hd0 + (hl // 16) * 16, 16)
            vhi = jnp.full((16,), clamp_hi[chunk][hl % 16], jnp.float32)
            vlo = jnp.full((16,), clamp_lo[chunk][hl % 16], jnp.float32)

            def clamp_fill(m, carry, hl=hl, vhi=vhi, vlo=vlo):
                gown[pl.ds(hl * S_ROWS + 16 * m, 16)] = vhi
                gown[pl.ds(hl * S_ROWS + 640 + 16 * m, 16)] = vlo
                return carry

            lax.fori_loop(0, 24, clamp_fill, 0)

        for hl in range(HDB):
            chunk2 = pl.ds(hd0 + (hl // 16) * 16, 16)
            vhi2 = jnp.full((16,), clamp_hi[chunk2][hl % 16], jnp.float32)

            def cfill_hi(m, carry, hl=hl, vhi2=vhi2):
                bufa[hl, pl.ds(16 * m, 16)] = vhi2
                return carry

            lax.fori_loop(0, L_STATIC // 16, cfill_hi, 0)
        pltpu.async_copy(bufa.at[pl.ds(0, HDB), pl.ds(0, 256)],
                         cl_sh.at[pl.ds(hd0, HDB), :], sems).wait()
        for hl in range(HDB):
            chunk2 = pl.ds(hd0 + (hl // 16) * 16, 16)
            vlo2 = jnp.full((16,), clamp_lo[chunk2][hl % 16], jnp.float32)

            def cfill_lo(m, carry, hl=hl, vlo2=vlo2):
                bufa[hl, pl.ds(16 * m, 16)] = vlo2
                return carry

            lax.fori_loop(0, L_STATIC // 16, cfill_lo, 0)
        pltpu.async_copy(bufa.at[pl.ds(0, HDB), pl.ds(0, 256)],
                         cr_sh.at[pl.ds(hd0, HDB), :], sems).wait()
        plsc.subcore_barrier()

        blocks = (
            (0, 256, cr_sh, 256, 256),
            (0, 384, cr_sh, 384, 128),
            (128, 384, cl_sh, 0, 128),
            (256, 256, cl_sh, 0, 256),
        )
        ring = ((bufa, sema), (bufb, semb), (bufc, semc), (bufd, semd))

        for blk, (mid_lo, mid_w, csrc, c_lo, c_w) in enumerate(blocks):
            ibase = blk * 128
            nm = mid_w // 16

            def emit(t, carry, ibase=ibase, mid_lo=mid_lo, mid_w=mid_w,
                     csrc=csrc, c_lo=c_lo, c_w=c_w, nm=nm):
                for par, (buf, sem) in enumerate(ring):
                    i = ibase + 4 * t + par
                    o = 511 - i

                    @pl.when(t > 0)
                    def _(buf=buf, sem=sem, mid_lo=mid_lo, mid_w=mid_w):
                        pltpu.make_async_copy(
                            buf.at[pl.ds(0, HDB), pl.ds(mid_lo, mid_w)],
                            out.at[pl.ds(hd0, HDB), pl.ds(mid_lo, mid_w)],
                            sem).wait()

                    @plsc.parallel_loop(0, HDB * nm, unroll=16)
                    def fill_row(q, buf=buf, o=o, mid_lo=mid_lo, nm=nm):
                        hl = q // nm
                        mq = q % nm
                        buf[hl, pl.ds(mid_lo + 16 * mq, 16)] = (
                            gown[pl.ds(hl * S_ROWS + o + mid_lo + 16 * mq, 16)])

                    pltpu.async_copy(
                        buf.at[pl.ds(0, HDB), pl.ds(mid_lo, mid_w)],
                        out.at[pl.ds(i * D + hd0, HDB), pl.ds(mid_lo, mid_w)],
                        sem)

                    @pl.when(i % NS == sid)
                    def _(i=i, csrc=csrc, c_lo=c_lo, c_w=c_w):
                        pltpu.async_copy(
                            csrc.at[pl.ds(0, D), pl.ds(0, c_w)],
                            out.at[pl.ds(i * D, D), pl.ds(c_lo, c_w)],
                            semc2)
                return carry

            lax.fori_loop(0, 32, emit, 0)
            for buf, sem in ring:
                pltpu.make_async_copy(
                    buf.at[pl.ds(0, HDB), pl.ds(mid_lo, mid_w)],
                    out.at[pl.ds(hd0, HDB), pl.ds(mid_lo, mid_w)], sem).wait()
            for _ in range(128 // NS):
                pltpu.make_async_copy(
                    csrc.at[pl.ds(0, D), pl.ds(0, c_w)],
                    out.at[pl.ds(0, D), pl.ds(c_lo, c_w)], semc2).wait()

    @pl.when(cid == 0)
    def _():
        run(tk_hbm, outk_hbm)

    @pl.when(cid != 0)
    def _():
        run(tv_hbm, outv_hbm)


@jax.jit
def _rel_emb(tk, tv):
    mesh = plsc.VectorSubcoreMesh(core_axis_name="c", subcore_axis_name="s")
    f = functools.partial(
        pl.kernel,
        mesh=mesh,
        out_type=(
            jax.ShapeDtypeStruct((L_STATIC * D, L_STATIC), jnp.float32),
            jax.ShapeDtypeStruct((L_STATIC * D, L_STATIC), jnp.float32),
        ),
        scratch_types=[
            pltpu.VMEM((HDB * 2, 128), jnp.int32),
            pltpu.VMEM((HDB * S_ROWS,), jnp.float32),
            pltpu.VMEM((D,), jnp.float32),
            pltpu.VMEM((D,), jnp.float32),
            pltpu.VMEM((HDB, L_STATIC), jnp.float32),
            pltpu.VMEM((HDB, L_STATIC), jnp.float32),
            pltpu.VMEM((HDB, L_STATIC), jnp.float32),
            pltpu.VMEM((HDB, L_STATIC), jnp.float32),
            pltpu.VMEM_SHARED((D, 256), jnp.float32),
            pltpu.VMEM_SHARED((D, 256), jnp.float32),
            pltpu.SemaphoreType.DMA,
            pltpu.SemaphoreType.DMA,
            pltpu.SemaphoreType.DMA,
            pltpu.SemaphoreType.DMA,
            pltpu.SemaphoreType.DMA,
            pltpu.SemaphoreType.DMA,
        ],
    )(_sc_body)
    return f(tk, tv)


def kernel(rel_pos_emb_k, rel_pos_emb_v, L):
    tk = rel_pos_emb_k.reshape(-1)
    tv = rel_pos_emb_v.reshape(-1)
    outk, outv = _rel_emb(tk, tv)
    outk = outk.reshape(L_STATIC, NUM_HEADS, D_K, L_STATIC).transpose(0, 3, 1, 2)
    outv = outv.reshape(L_STATIC, NUM_HEADS, D_K, L_STATIC).transpose(0, 3, 1, 2)
    return (outk, outv)
